# baseline (speedup 1.0000x reference)
import functools

import jax
import jax.numpy as jnp
from jax import lax
from jax.experimental import pallas as pl
from jax.experimental.pallas import tpu as pltpu
from jax.experimental.pallas import tpu_sc as plsc

N_NODES = 10000
D_FEAT = 256
DH = 128
N_EDGES = 160000
NSUB = 16
B = 80
EPW = N_EDGES // NSUB
CHUNKS = EPW // B
RPW = 624
TAIL = N_NODES - RPW * NSUB
TAIL0 = RPW * NSUB
LANES = 16


HB = B // 2


def _gine_sc(feat, src, dst, efeat, out,
             sidx, didxa, didxb, gatha, gathb, ebuf, acc,
             sem_g0, sem_g1, sem_e0, sem_e1, sem_da0, sem_da1,
             sem_db0, sem_db1, sem_s0, sem_s1, sem_i0, sem_i1):
    cid = lax.axis_index("c")
    sid = lax.axis_index("s")
    e0 = sid * EPW
    coff = pl.multiple_of(cid * DH, DH)

    sem_g = (sem_g0, sem_g1)
    sem_e = (sem_e0, sem_e1)
    sem_da = (sem_da0, sem_da1)
    sem_db = (sem_db0, sem_db1)
    sem_s = (sem_s0, sem_s1)
    sem_i = (sem_i0, sem_i1)

    def start(g, b, first=False):
        base = pl.multiple_of(e0 + g * B, B)
        if first:
            pltpu.sync_copy(src.at[pl.ds(base, B)], sidx[b])
        else:
            pltpu.make_async_copy(src.at[pl.ds(0, B)], sidx[b],
                                  sem_i[b]).wait()
        pltpu.async_copy(efeat.at[pl.ds(base, B), pl.ds(coff, DH)],
                         ebuf[b], sem_e[b])
        if not first:
            pltpu.make_async_copy(gatha[b], acc.at[didxa[b]],
                                  sem_s[b]).wait()
            pltpu.make_async_copy(gathb[b], acc.at[didxb[b]],
                                  sem_s[b]).wait()
        pltpu.async_copy(dst.at[pl.ds(base, HB)], didxa[b], sem_da[b])
        pltpu.async_copy(dst.at[pl.ds(base + HB, HB)], didxb[b], sem_db[b])
        pltpu.async_copy(feat.at[sidx[b].at[pl.ds(0, HB)], pl.ds(coff, DH)],
                         gatha[b], sem_g[b])
        pltpu.async_copy(feat.at[sidx[b].at[pl.ds(HB, HB)], pl.ds(coff, DH)],
                         gathb[b], sem_g[b])

    def finish(g, b, load_next=True):
        pltpu.make_async_copy(feat.at[didxa[b], pl.ds(coff, DH)], gatha[b],
                              sem_g[b]).wait()
        pltpu.make_async_copy(feat.at[didxa[b], pl.ds(coff, DH)], gathb[b],
                              sem_g[b]).wait()
        if load_next:
            nbase = pl.multiple_of(e0 + (g + 2) * B, B)
            pltpu.async_copy(src.at[pl.ds(nbase, B)], sidx[b], sem_i[b])
        pltpu.make_async_copy(efeat.at[pl.ds(0, B), pl.ds(coff, DH)],
                              ebuf[b], sem_e[b]).wait()

        def crow_a(r, c2):
            for j in range(DH // LANES):
                s_ = pl.ds(j * LANES, LANES)
                gatha[b][r, s_] = jnp.maximum(
                    gatha[b][r, s_] + ebuf[b][r, s_], 0.0)
            return c2

        def crow_b(r, c2):
            for j in range(DH // LANES):
                s_ = pl.ds(j * LANES, LANES)
                gathb[b][r, s_] = jnp.maximum(
                    gathb[b][r, s_] + ebuf[b][r + HB, s_], 0.0)
            return c2

        lax.fori_loop(0, HB, crow_a, 0, unroll=2)
        pltpu.make_async_copy(dst.at[pl.ds(0, HB)], didxa[b],
                              sem_da[b]).wait()
        pltpu.async_copy(gatha[b], acc.at[didxa[b]], sem_s[b], add=True)
        lax.fori_loop(0, HB, crow_b, 0, unroll=2)
        pltpu.make_async_copy(dst.at[pl.ds(0, HB)], didxb[b],
                              sem_db[b]).wait()
        pltpu.async_copy(gathb[b], acc.at[didxb[b]], sem_s[b], add=True)

    start(0, 0, first=True)
    start(1, 1, first=True)

    r0 = sid * RPW
    pltpu.sync_copy(feat.at[pl.ds(r0, RPW), pl.ds(coff, DH)],
                    acc.at[pl.ds(r0, RPW)])

    @pl.when(sid == NSUB - 1)
    def _():
        pltpu.sync_copy(feat.at[pl.ds(TAIL0, TAIL), pl.ds(coff, DH)],
                        acc.at[pl.ds(TAIL0, TAIL)])

    plsc.subcore_barrier()

    def pair(k, carry):
        g = k * 2
        finish(g, 0)
        start(g + 2, 0)
        finish(g + 1, 1)
        start(g + 3, 1)
        return carry

    lax.fori_loop(0, (CHUNKS - 3) // 2, pair, 0, unroll=False)
    finish(CHUNKS - 3, 0)
    start(CHUNKS - 1, 0)
    finish(CHUNKS - 2, 1, load_next=False)
    finish(CHUNKS - 1, 0, load_next=False)

    for b in (0, 1):
        pltpu.make_async_copy(gatha[b], acc.at[didxa[b]], sem_s[b]).wait()
        pltpu.make_async_copy(gathb[b], acc.at[didxb[b]], sem_s[b]).wait()

    plsc.subcore_barrier()

    pltpu.sync_copy(acc.at[pl.ds(r0, RPW)],
                    out.at[pl.ds(r0, RPW), pl.ds(coff, DH)])

    @pl.when(sid == NSUB - 1)
    def _():
        pltpu.sync_copy(acc.at[pl.ds(TAIL0, TAIL)],
                        out.at[pl.ds(TAIL0, TAIL), pl.ds(coff, DH)])


_mesh = plsc.VectorSubcoreMesh(core_axis_name="c", subcore_axis_name="s")

_gine_call = functools.partial(
    pl.kernel,
    out_type=jax.ShapeDtypeStruct((N_NODES, D_FEAT), jnp.float32),
    mesh=_mesh,
    scratch_types=[
        [pltpu.VMEM((B,), jnp.int32)] * 2,
        [pltpu.VMEM((HB,), jnp.int32)] * 2,
        [pltpu.VMEM((HB,), jnp.int32)] * 2,
        [pltpu.VMEM((HB, DH), jnp.float32)] * 2,
        [pltpu.VMEM((HB, DH), jnp.float32)] * 2,
        [pltpu.VMEM((B, DH), jnp.float32)] * 2,
        pltpu.VMEM_SHARED((N_NODES, DH), jnp.float32),
    ] + [pltpu.SemaphoreType.DMA] * 12,
)(_gine_sc)


@jax.jit
def kernel(feat, edge_index, efeat):
    src = edge_index[0].astype(jnp.int32)
    dst = edge_index[1].astype(jnp.int32)
    return _gine_call(feat, src, dst, efeat)


# speedup vs baseline: 2.4333x; 2.4333x over previous
import functools

import jax
import jax.numpy as jnp
from jax import lax
from jax.experimental import pallas as pl
from jax.experimental.pallas import tpu as pltpu
from jax.experimental.pallas import tpu_sc as plsc

N_NODES = 10000
D_FEAT = 256
DH = 128
N_EDGES = 160000
NSUB = 16
B = 80
EPW = N_EDGES // NSUB
CHUNKS = EPW // B
RPW = 624
TAIL = N_NODES - RPW * NSUB
TAIL0 = RPW * NSUB
LANES = 16


HB = B // 2


def _gine_sc(feat, src, dst, efeat, out,
             sidx, didxa, didxb, gatha, gathb, ebuf, acc,
             sem_g0, sem_g1, sem_e0, sem_e1, sem_da0, sem_da1,
             sem_db0, sem_db1, sem_s0, sem_s1, sem_i0, sem_i1):
    cid = lax.axis_index("c")
    sid = lax.axis_index("s")
    e0 = sid * EPW
    coff = pl.multiple_of(cid * DH, DH)

    sem_g = (sem_g0, sem_g1)
    sem_e = (sem_e0, sem_e1)
    sem_da = (sem_da0, sem_da1)
    sem_db = (sem_db0, sem_db1)
    sem_s = (sem_s0, sem_s1)
    sem_i = (sem_i0, sem_i1)

    def start(g, b, first=False):
        base = pl.multiple_of(e0 + g * B, B)
        if first:
            pltpu.sync_copy(src.at[pl.ds(base, B)], sidx[b])
        else:
            pltpu.make_async_copy(src.at[pl.ds(0, B)], sidx[b],
                                  sem_i[b]).wait()
        pltpu.async_copy(efeat.at[pl.ds(base, B), pl.ds(coff, DH)],
                         ebuf[b], sem_e[b])
        if not first:
            pltpu.make_async_copy(gatha[b], acc.at[didxa[b]],
                                  sem_s[b]).wait()
            pltpu.make_async_copy(gathb[b], acc.at[didxb[b]],
                                  sem_s[b]).wait()
        pltpu.async_copy(dst.at[pl.ds(base, HB)], didxa[b], sem_da[b])
        pltpu.async_copy(dst.at[pl.ds(base + HB, HB)], didxb[b], sem_db[b])
        pltpu.async_copy(feat.at[sidx[b].at[pl.ds(0, HB)], pl.ds(coff, DH)],
                         gatha[b], sem_g[b])
        pltpu.async_copy(feat.at[sidx[b].at[pl.ds(HB, HB)], pl.ds(coff, DH)],
                         gathb[b], sem_g[b])

    def finish(g, b, load_next=True):
        pltpu.make_async_copy(feat.at[didxa[b], pl.ds(coff, DH)], gatha[b],
                              sem_g[b]).wait()
        pltpu.make_async_copy(feat.at[didxa[b], pl.ds(coff, DH)], gathb[b],
                              sem_g[b]).wait()
        if load_next:
            nbase = pl.multiple_of(e0 + (g + 2) * B, B)
            pltpu.async_copy(src.at[pl.ds(nbase, B)], sidx[b], sem_i[b])
        pltpu.make_async_copy(efeat.at[pl.ds(0, B), pl.ds(coff, DH)],
                              ebuf[b], sem_e[b]).wait()

        def crow_a(r, c2):
            for j in range(DH // LANES):
                s_ = pl.ds(j * LANES, LANES)
                gatha[b][r, s_] = jnp.maximum(
                    gatha[b][r, s_] + ebuf[b][r, s_], 0.0)
            return c2

        def crow_b(r, c2):
            for j in range(DH // LANES):
                s_ = pl.ds(j * LANES, LANES)
                gathb[b][r, s_] = jnp.maximum(
                    gathb[b][r, s_] + ebuf[b][r + HB, s_], 0.0)
            return c2

        lax.fori_loop(0, HB, crow_a, 0, unroll=False)
        pltpu.make_async_copy(dst.at[pl.ds(0, HB)], didxa[b],
                              sem_da[b]).wait()
        pltpu.async_copy(gatha[b], acc.at[didxa[b]], sem_s[b], add=True)
        lax.fori_loop(0, HB, crow_b, 0, unroll=False)
        pltpu.make_async_copy(dst.at[pl.ds(0, HB)], didxb[b],
                              sem_db[b]).wait()
        pltpu.async_copy(gathb[b], acc.at[didxb[b]], sem_s[b], add=True)

    start(0, 0, first=True)
    start(1, 1, first=True)

    r0 = sid * RPW
    pltpu.sync_copy(feat.at[pl.ds(r0, RPW), pl.ds(coff, DH)],
                    acc.at[pl.ds(r0, RPW)])

    @pl.when(sid == NSUB - 1)
    def _():
        pltpu.sync_copy(feat.at[pl.ds(TAIL0, TAIL), pl.ds(coff, DH)],
                        acc.at[pl.ds(TAIL0, TAIL)])

    plsc.subcore_barrier()

    def pair(k, carry):
        g = k * 2
        finish(g, 0)
        start(g + 2, 0)
        finish(g + 1, 1)
        start(g + 3, 1)
        return carry

    lax.fori_loop(0, (CHUNKS - 3) // 2, pair, 0, unroll=False)
    finish(CHUNKS - 3, 0)
    start(CHUNKS - 1, 0)
    finish(CHUNKS - 2, 1, load_next=False)
    finish(CHUNKS - 1, 0, load_next=False)

    for b in (0, 1):
        pltpu.make_async_copy(gatha[b], acc.at[didxa[b]], sem_s[b]).wait()
        pltpu.make_async_copy(gathb[b], acc.at[didxb[b]], sem_s[b]).wait()

    plsc.subcore_barrier()

    pltpu.sync_copy(acc.at[pl.ds(r0, RPW)],
                    out.at[pl.ds(r0, RPW), pl.ds(coff, DH)])

    @pl.when(sid == NSUB - 1)
    def _():
        pltpu.sync_copy(acc.at[pl.ds(TAIL0, TAIL)],
                        out.at[pl.ds(TAIL0, TAIL), pl.ds(coff, DH)])


_mesh = plsc.VectorSubcoreMesh(core_axis_name="c", subcore_axis_name="s")

_gine_call = functools.partial(
    pl.kernel,
    out_type=jax.ShapeDtypeStruct((N_NODES, D_FEAT), jnp.float32),
    mesh=_mesh,
    scratch_types=[
        [pltpu.VMEM((B,), jnp.int32)] * 2,
        [pltpu.VMEM((HB,), jnp.int32)] * 2,
        [pltpu.VMEM((HB,), jnp.int32)] * 2,
        [pltpu.VMEM((HB, DH), jnp.float32)] * 2,
        [pltpu.VMEM((HB, DH), jnp.float32)] * 2,
        [pltpu.VMEM((B, DH), jnp.float32)] * 2,
        pltpu.VMEM_SHARED((N_NODES, DH), jnp.float32),
    ] + [pltpu.SemaphoreType.DMA] * 12,
)(_gine_sc)


@jax.jit
def kernel(feat, edge_index, efeat):
    src = edge_index[0].astype(jnp.int32)
    dst = edge_index[1].astype(jnp.int32)
    return _gine_call(feat, src, dst, efeat)


# speedup vs baseline: 2.4546x; 1.0088x over previous
import functools

import jax
import jax.numpy as jnp
from jax import lax
from jax.experimental import pallas as pl
from jax.experimental.pallas import tpu as pltpu
from jax.experimental.pallas import tpu_sc as plsc

N_NODES = 10000
D_FEAT = 256
DH = 128
N_EDGES = 160000
NSUB = 16
B = 80
EPW = N_EDGES // NSUB
CHUNKS = EPW // B
RPW = 624
TAIL = N_NODES - RPW * NSUB
TAIL0 = RPW * NSUB
LANES = 16


HB = B // 2


def _gine_sc(feat, src, dst, efeat, out,
             sidx, didxa, didxb, gatha, gathb, ebuf, acc,
             sem_ga0, sem_ga1, sem_gb0, sem_gb1, sem_e0, sem_e1,
             sem_da0, sem_da1, sem_db0, sem_db1,
             sem_sa0, sem_sa1, sem_sb0, sem_sb1, sem_i0, sem_i1):
    cid = lax.axis_index("c")
    sid = lax.axis_index("s")
    e0 = sid * EPW
    coff = pl.multiple_of(cid * DH, DH)

    sem_ga = (sem_ga0, sem_ga1)
    sem_gb = (sem_gb0, sem_gb1)
    sem_e = (sem_e0, sem_e1)
    sem_da = (sem_da0, sem_da1)
    sem_db = (sem_db0, sem_db1)
    sem_sa = (sem_sa0, sem_sa1)
    sem_sb = (sem_sb0, sem_sb1)
    sem_i = (sem_i0, sem_i1)

    def start(g, b, first=False):
        base = pl.multiple_of(e0 + g * B, B)
        if first:
            pltpu.sync_copy(src.at[pl.ds(base, B)], sidx[b])
        else:
            pltpu.make_async_copy(src.at[pl.ds(0, B)], sidx[b],
                                  sem_i[b]).wait()
        pltpu.async_copy(efeat.at[pl.ds(base, B), pl.ds(coff, DH)],
                         ebuf[b], sem_e[b])
        if not first:
            pltpu.make_async_copy(gatha[b], acc.at[didxa[b]],
                                  sem_sa[b]).wait()
            pltpu.make_async_copy(gathb[b], acc.at[didxb[b]],
                                  sem_sb[b]).wait()
        pltpu.async_copy(dst.at[pl.ds(base, HB)], didxa[b], sem_da[b])
        pltpu.async_copy(dst.at[pl.ds(base + HB, HB)], didxb[b], sem_db[b])
        pltpu.async_copy(feat.at[sidx[b].at[pl.ds(0, HB)], pl.ds(coff, DH)],
                         gatha[b], sem_ga[b])
        pltpu.async_copy(feat.at[sidx[b].at[pl.ds(HB, HB)], pl.ds(coff, DH)],
                         gathb[b], sem_gb[b])

    def finish(g, b, load_next=True):
        pltpu.make_async_copy(feat.at[didxa[b], pl.ds(coff, DH)], gatha[b],
                              sem_ga[b]).wait()
        pltpu.make_async_copy(feat.at[didxa[b], pl.ds(coff, DH)], gathb[b],
                              sem_gb[b]).wait()
        if load_next:
            nbase = pl.multiple_of(e0 + (g + 2) * B, B)
            pltpu.async_copy(src.at[pl.ds(nbase, B)], sidx[b], sem_i[b])
        pltpu.make_async_copy(efeat.at[pl.ds(0, B), pl.ds(coff, DH)],
                              ebuf[b], sem_e[b]).wait()

        def crow_a(r, c2):
            for j in range(DH // LANES):
                s_ = pl.ds(j * LANES, LANES)
                gatha[b][r, s_] = jnp.maximum(
                    gatha[b][r, s_] + ebuf[b][r, s_], 0.0)
            return c2

        def crow_b(r, c2):
            for j in range(DH // LANES):
                s_ = pl.ds(j * LANES, LANES)
                gathb[b][r, s_] = jnp.maximum(
                    gathb[b][r, s_] + ebuf[b][r + HB, s_], 0.0)
            return c2

        lax.fori_loop(0, HB, crow_a, 0, unroll=False)
        pltpu.make_async_copy(dst.at[pl.ds(0, HB)], didxa[b],
                              sem_da[b]).wait()
        pltpu.async_copy(gatha[b], acc.at[didxa[b]], sem_sa[b], add=True)
        lax.fori_loop(0, HB, crow_b, 0, unroll=False)
        pltpu.make_async_copy(dst.at[pl.ds(0, HB)], didxb[b],
                              sem_db[b]).wait()
        pltpu.async_copy(gathb[b], acc.at[didxb[b]], sem_sb[b], add=True)

    start(0, 0, first=True)
    start(1, 1, first=True)

    r0 = sid * RPW
    pltpu.sync_copy(feat.at[pl.ds(r0, RPW), pl.ds(coff, DH)],
                    acc.at[pl.ds(r0, RPW)])

    @pl.when(sid == NSUB - 1)
    def _():
        pltpu.sync_copy(feat.at[pl.ds(TAIL0, TAIL), pl.ds(coff, DH)],
                        acc.at[pl.ds(TAIL0, TAIL)])

    plsc.subcore_barrier()

    def pair(k, carry):
        g = k * 2
        finish(g, 0)
        start(g + 2, 0)
        finish(g + 1, 1)
        start(g + 3, 1)
        return carry

    lax.fori_loop(0, (CHUNKS - 3) // 2, pair, 0, unroll=False)
    finish(CHUNKS - 3, 0)
    start(CHUNKS - 1, 0)
    finish(CHUNKS - 2, 1, load_next=False)
    finish(CHUNKS - 1, 0, load_next=False)

    for b in (0, 1):
        pltpu.make_async_copy(gatha[b], acc.at[didxa[b]], sem_sa[b]).wait()
        pltpu.make_async_copy(gathb[b], acc.at[didxb[b]], sem_sb[b]).wait()

    plsc.subcore_barrier()

    pltpu.sync_copy(acc.at[pl.ds(r0, RPW)],
                    out.at[pl.ds(r0, RPW), pl.ds(coff, DH)])

    @pl.when(sid == NSUB - 1)
    def _():
        pltpu.sync_copy(acc.at[pl.ds(TAIL0, TAIL)],
                        out.at[pl.ds(TAIL0, TAIL), pl.ds(coff, DH)])


_mesh = plsc.VectorSubcoreMesh(core_axis_name="c", subcore_axis_name="s")

_gine_call = functools.partial(
    pl.kernel,
    out_type=jax.ShapeDtypeStruct((N_NODES, D_FEAT), jnp.float32),
    mesh=_mesh,
    scratch_types=[
        [pltpu.VMEM((B,), jnp.int32)] * 2,
        [pltpu.VMEM((HB,), jnp.int32)] * 2,
        [pltpu.VMEM((HB,), jnp.int32)] * 2,
        [pltpu.VMEM((HB, DH), jnp.float32)] * 2,
        [pltpu.VMEM((HB, DH), jnp.float32)] * 2,
        [pltpu.VMEM((B, DH), jnp.float32)] * 2,
        pltpu.VMEM_SHARED((N_NODES, DH), jnp.float32),
    ] + [pltpu.SemaphoreType.DMA] * 16,
)(_gine_sc)


@jax.jit
def kernel(feat, edge_index, efeat):
    src = edge_index[0].astype(jnp.int32)
    dst = edge_index[1].astype(jnp.int32)
    return _gine_call(feat, src, dst, efeat)
